# baseline (speedup 1.0000x reference)
import functools

import jax
import jax.numpy as jnp
from jax import lax
from jax.experimental import pallas as pl
from jax.experimental.pallas import tpu as pltpu
from jax.experimental.pallas import tpu_sc as plsc

D = 128
NC = 2
NS = 16
NW = NC * NS
RING = 6
GLAG = 2
ROWS = 5000
BLOCK = 160
BLOCK_PAD = BLOCK * NW
SLOTS = -(-(BLOCK + RING) // RING) * RING


def _gather_body(x_hbm, idx_hbm, out_hbm, idx_v, bufs, *sems):
    gsem, wsem = sems[:RING], sems[RING:]
    wid = lax.axis_index("s") * NC + lax.axis_index("c")
    base = wid * BLOCK

    pltpu.sync_copy(idx_hbm.at[pl.ds(base, BLOCK)], idx_v)

    def fire_gather(ch, b):
        pltpu.async_copy(x_hbm.at[idx_v.at[ch]], bufs.at[b], gsem[b])

    def drain_gather(b):
        pltpu.make_async_copy(x_hbm.at[pl.ds(0, D)], bufs.at[b], gsem[b]).wait()

    def fire_write(ch, b):
        pltpu.async_copy(bufs.at[b], out_hbm.at[pl.ds((base + ch) * D, D)], wsem[b])

    def drain_write(b):
        pltpu.make_async_copy(bufs.at[b], out_hbm.at[pl.ds(0, D)], wsem[b]).wait()

    @pl.loop(0, SLOTS, step=RING)
    def _slot(s0):
        for b in range(RING):
            s = s0 + b
            cg = s - GLAG
            cw = s - RING

            @pl.when((s >= RING) & (cw < BLOCK) & (base + cw < ROWS))
            def _():
                drain_write(b)

            @pl.when(s < BLOCK)
            def _():
                fire_gather(s, b)

            bg = (b - GLAG) % RING

            @pl.when((s >= GLAG) & (cg < BLOCK))
            def _():
                drain_gather(bg)

                @pl.when(base + cg < ROWS)
                def _():
                    fire_write(cg, bg)


@jax.jit
def kernel(x, edge_index):
    n_edges = edge_index.shape[1]
    idx2 = jnp.transpose(edge_index).reshape(ROWS, D)
    idx2 = jnp.zeros((BLOCK_PAD, D), jnp.int32).at[:ROWS].set(idx2)
    mesh = plsc.VectorSubcoreMesh(
        core_axis_name="c", subcore_axis_name="s", num_cores=NC, num_subcores=NS
    )
    run = pl.kernel(
        _gather_body,
        out_type=jax.ShapeDtypeStruct((2 * n_edges, D), jnp.float32),
        mesh=mesh,
        scratch_types=[
            pltpu.VMEM((BLOCK, D), jnp.int32),
            pltpu.VMEM((RING, D, D), jnp.float32),
        ] + [pltpu.SemaphoreType.DMA] * (2 * RING),
    )
    out = run(x, idx2)
    return out.reshape(n_edges, 2 * D)


# speedup vs baseline: 1.1752x; 1.1752x over previous
import jax
import jax.numpy as jnp
from jax import lax
from jax.experimental import pallas as pl
from jax.experimental.pallas import tpu as pltpu
from jax.experimental.pallas import tpu_sc as plsc

D = 128
NC = 2
NS = 16
NW = NC * NS
RING = 6
GLAG = 2
ROWS = 5000
BLOCK = 160
ROWS_PAD = BLOCK * NW
SLOTS = -(-(BLOCK + RING) // RING) * RING


def _gather_body(x_hbm, idx_hbm, out_hbm, idx_v, bufs, *sems):
    gsem, wsem = sems[:RING], sems[RING:]
    wid = lax.axis_index("s") * NC + lax.axis_index("c")

    pltpu.sync_copy(idx_hbm.at[wid], idx_v)

    @pl.loop(0, SLOTS, step=RING)
    def _slot(s0):
        for b in range(RING):
            s = s0 + b
            cg = s - GLAG
            cw = s - RING

            @pl.when((s >= RING) & (cw * NW + wid < ROWS))
            def _():
                pltpu.make_async_copy(
                    bufs.at[b], out_hbm.at[pl.ds(0, D)], wsem[b]
                ).wait()

            @pl.when(s < BLOCK)
            def _():
                pltpu.async_copy(x_hbm.at[idx_v.at[s]], bufs.at[b], gsem[b])

            bg = (b - GLAG) % RING

            @pl.when((s >= GLAG) & (cg < BLOCK))
            def _():
                pltpu.make_async_copy(
                    x_hbm.at[pl.ds(0, D)], bufs.at[bg], gsem[bg]
                ).wait()

                @pl.when(cg * NW + wid < ROWS)
                def _():
                    pltpu.async_copy(
                        bufs.at[bg],
                        out_hbm.at[pl.ds((cg * NW + wid) * D, D)],
                        wsem[bg],
                    )


@jax.jit
def kernel(x, edge_index):
    n_edges = edge_index.shape[1]
    idx2 = jnp.transpose(edge_index).reshape(ROWS, D)
    idx2 = jnp.zeros((ROWS_PAD, D), jnp.int32).at[:ROWS].set(idx2)
    idx3 = jnp.transpose(idx2.reshape(BLOCK, NW, D), (1, 0, 2))
    mesh = plsc.VectorSubcoreMesh(
        core_axis_name="c", subcore_axis_name="s", num_cores=NC, num_subcores=NS
    )
    run = pl.kernel(
        _gather_body,
        out_type=jax.ShapeDtypeStruct((2 * n_edges, D), jnp.float32),
        mesh=mesh,
        scratch_types=[
            pltpu.VMEM((BLOCK, D), jnp.int32),
            pltpu.VMEM((RING, D, D), jnp.float32),
        ] + [pltpu.SemaphoreType.DMA] * (2 * RING),
    )
    out = run(x, idx3)
    return out.reshape(n_edges, 2 * D)
